# baseline (speedup 1.0000x reference)
import functools

import jax
import jax.numpy as jnp
from jax.experimental import pallas as pl
from jax.experimental.pallas import tpu as pltpu


def _manual_kernel(x_hbm, B_ref, b_ref, o_hbm, xb, ob, in_sems, out_sems,
                   *, T, BN, CV):
    i = pl.program_id(0)
    slot = jax.lax.rem(i, 2)
    nslot = jax.lax.rem(i + 1, 2)
    Cout, V, L = ob.shape[2], ob.shape[3], ob.shape[4]

    def copy_in(step, sl):
        return pltpu.make_async_copy(
            x_hbm.at[pl.ds(step * BN, BN)], xb.at[sl], in_sems.at[sl])

    def copy_out(step, sl):
        return pltpu.make_async_copy(
            ob.at[sl], o_hbm.at[pl.ds(step * BN, BN)], out_sems.at[sl])

    @pl.when(i == 0)
    def _():
        copy_in(0, 0).start()

    @pl.when(i + 1 < T)
    def _():
        copy_in(i + 1, nslot).start()

    copy_in(i, slot).wait()

    @pl.when(i >= 2)
    def _():
        copy_out(i - 2, slot).wait()

    for j in range(BN):
        xj = xb[slot, j].reshape(CV, L).astype(jnp.bfloat16)
        acc = jnp.dot(B_ref[...], xj, preferred_element_type=jnp.float32)
        acc = acc.reshape(Cout, V, L) + b_ref[...][:, :, None]
        ob[slot, j] = acc.astype(ob.dtype)

    copy_out(i, slot).start()

    @pl.when(i == T - 1)
    def _():
        copy_out(i - 1, nslot).wait()
        copy_out(i, slot).wait()


def _fold_weights(support, W, C, V):
    S = support.shape[0]
    Cout, Ctot = W.shape[0], W.shape[1]
    order = (Ctot // C - 1) // S
    mats = [jnp.eye(V, dtype=jnp.float32)]
    for s in range(S):
        At = jnp.transpose(support[s]).astype(jnp.float32)
        Mk = jnp.eye(V, dtype=jnp.float32)
        for _ in range(order):
            Mk = jnp.dot(At, Mk)
            mats.append(Mk)
    Ms = jnp.stack(mats, 0)
    Wb = W.reshape(Cout, Ms.shape[0], C).astype(jnp.float32)
    B = jnp.einsum('obc,bvw->ovcw', Wb, Ms)
    return B.reshape(Cout * V, C * V)


def kernel(x, support, W, b):
    N, C, V, L = x.shape
    Cout = W.shape[0]
    CV = C * V

    B = _fold_weights(support, W, C, V).astype(jnp.bfloat16)
    b2 = b.reshape(Cout, 1).astype(jnp.float32)

    BN = 8 if N % 8 == 0 else 1
    T = N // BN

    flops = 2 * (Cout * V) * CV * N * L
    bytes_accessed = 4 * (N * C * V * L + N * Cout * V * L) + 2 * Cout * V * CV

    kernel_fn = functools.partial(_manual_kernel, T=T, BN=BN, CV=CV)
    out = pl.pallas_call(
        kernel_fn,
        out_shape=jax.ShapeDtypeStruct((N, Cout, V, L), x.dtype),
        grid=(T,),
        in_specs=[
            pl.BlockSpec(memory_space=pl.ANY),
            pl.BlockSpec((Cout * V, CV), lambda t: (0, 0)),
            pl.BlockSpec((Cout, 1), lambda t: (0, 0)),
        ],
        out_specs=pl.BlockSpec(memory_space=pl.ANY),
        scratch_shapes=[
            pltpu.VMEM((2, BN, C, V, L), jnp.float32),
            pltpu.VMEM((2, BN, Cout, V, L), jnp.float32),
            pltpu.SemaphoreType.DMA((2,)),
            pltpu.SemaphoreType.DMA((2,)),
        ],
        compiler_params=pltpu.CompilerParams(
            dimension_semantics=("arbitrary",)),
        cost_estimate=pl.CostEstimate(flops=int(flops), transcendentals=0,
                                      bytes_accessed=int(bytes_accessed)),
    )(x, B, b2)
    return out


# speedup vs baseline: 1.2314x; 1.2314x over previous
import functools

import jax
import jax.numpy as jnp
from jax.experimental import pallas as pl
from jax.experimental.pallas import tpu as pltpu


def _gcn_kernel(x_ref, sup_ref, w_ref, b_ref, o_ref, B_ref, *,
                C, V, S, order, BN):
    Cout, TL = o_ref.shape[1], o_ref.shape[3]
    CV = C * V

    @pl.when(pl.program_id(0) == 0)
    def _build_folded_weight():
        rows = jax.lax.broadcasted_iota(jnp.int32, (V, V), 0)
        cols = jax.lax.broadcasted_iota(jnp.int32, (V, V), 1)
        mats = [(rows == cols).astype(jnp.float32)]
        for s in range(S):
            a_t = sup_ref[s].T
            m_k = mats[0]
            for _ in range(order):
                m_k = jnp.dot(a_t, m_k, preferred_element_type=jnp.float32)
                mats.append(m_k)
        for c in range(C):
            acc = None
            for blk, m in enumerate(mats):
                wcol = w_ref[:, blk * C + c][:, None, None]
                term = wcol * m[None, :, :]
                acc = term if acc is None else acc + term
            B_ref[:, c * V:(c + 1) * V] = acc.reshape(Cout * V, V).astype(
                B_ref.dtype)

    for j in range(BN):
        xj = x_ref[j].reshape(CV, TL).astype(jnp.bfloat16)
        acc = jnp.dot(B_ref[...], xj, preferred_element_type=jnp.float32)
        acc = acc.reshape(Cout, V, TL) + b_ref[...][:, :, None]
        o_ref[j] = acc.astype(o_ref.dtype)


def kernel(x, support, W, b):
    N, C, V, L = x.shape
    S = support.shape[0]
    Cout, Ctot = W.shape[0], W.shape[1]
    order = (Ctot // C - 1) // S
    CV = C * V

    w2 = W.reshape(Cout, Ctot).astype(jnp.float32)
    b2 = b.reshape(Cout, 1).astype(jnp.float32)

    BN = 8 if N % 8 == 0 else 1
    T = N // BN

    flops = 2 * (Cout * V) * CV * N * L
    bytes_accessed = 4 * (N * C * V * L + N * Cout * V * L)

    kernel_fn = functools.partial(_gcn_kernel, C=C, V=V, S=S, order=order,
                                  BN=BN)
    out = pl.pallas_call(
        kernel_fn,
        out_shape=jax.ShapeDtypeStruct((N, Cout, V, L), x.dtype),
        grid=(T,),
        in_specs=[
            pl.BlockSpec((BN, C, V, L), lambda t: (t, 0, 0, 0)),
            pl.BlockSpec((S, V, V), lambda t: (0, 0, 0)),
            pl.BlockSpec((Cout, Ctot), lambda t: (0, 0)),
            pl.BlockSpec((Cout, 1), lambda t: (0, 0)),
        ],
        out_specs=pl.BlockSpec((BN, Cout, V, L), lambda t: (t, 0, 0, 0)),
        scratch_shapes=[pltpu.VMEM((Cout * V, CV), jnp.bfloat16)],
        compiler_params=pltpu.CompilerParams(
            dimension_semantics=("arbitrary",)),
        cost_estimate=pl.CostEstimate(flops=int(flops), transcendentals=0,
                                      bytes_accessed=int(bytes_accessed)),
    )(x, support.astype(jnp.float32), w2, b2)
    return out
